# Initial kernel scaffold; baseline (speedup 1.0000x reference)
import jax
import jax.numpy as jnp
from jax.experimental import pallas as pl


def kernel(x, x_fc, is_alive, graph_in, graph_out, edge_index_short, edge_index_log, params):
    raise NotImplementedError("write your pallas kernel here")


# speedup vs baseline: 14.4944x; 14.4944x over previous
import functools
import jax
import jax.numpy as jnp
from jax import lax
from jax.experimental import pallas as pl
from jax.experimental.pallas import tpu as pltpu
from jax.experimental.pallas import tpu_sc as plsc

CIN = 16
COUT = 64
NST = 512
BATCH = 32
NG = 7
NFLAT = BATCH * NST
NSUB = 16
NCORE = 2


def _sc_degree(dst2, zeros_pad, ones_pad):
    nrows = dst2.shape[0]
    rows_per_sub = nrows // (NCORE * NSUB)
    nf_sub = NFLAT // NSUB
    mesh = plsc.VectorSubcoreMesh(
        core_axis_name="c", subcore_axis_name="s", num_cores=NCORE, num_subcores=NSUB
    )

    @functools.partial(
        pl.kernel,
        out_type=jax.ShapeDtypeStruct((NCORE, NFLAT, 8), jnp.float32),
        mesh=mesh,
        scratch_types=[
            pltpu.VMEM((rows_per_sub, 128), jnp.int32),
            pltpu.VMEM((128, 8), jnp.float32),
            pltpu.VMEM_SHARED((NFLAT, 8), jnp.float32),
        ],
        compiler_params=pltpu.CompilerParams(use_tc_tiling_on_sc=False),
    )
    def k(dst_hbm, z_hbm, one_hbm, out_hbm, idx_v, ones_v, acc):
        c = lax.axis_index("c")
        s = lax.axis_index("s")
        pltpu.sync_copy(
            z_hbm.at[pl.ds(s * nf_sub, nf_sub)],
            acc.at[pl.ds(s * nf_sub, nf_sub)],
        )
        pltpu.sync_copy(one_hbm, ones_v)
        base = (c * NSUB + s) * rows_per_sub
        pltpu.sync_copy(dst_hbm.at[pl.ds(base, rows_per_sub)], idx_v)
        plsc.subcore_barrier()

        def step(j, carry):
            pltpu.sync_copy(ones_v, acc.at[idx_v.at[j]], add=True)
            return carry

        lax.fori_loop(0, rows_per_sub, step, 0)
        plsc.subcore_barrier()
        pltpu.sync_copy(
            acc.at[pl.ds(s * nf_sub, nf_sub)],
            out_hbm.at[c, pl.ds(s * nf_sub, nf_sub)],
        )

    return k(dst2, zeros_pad, ones_pad)


def _sc_gcn_aggregate(u2, src2, dst2):
    nrows = src2.shape[0]
    rows_per_sub = nrows // NSUB
    nf_sub = NFLAT // NSUB
    nbuf = 2
    nbrows = 32
    nblk = rows_per_sub // nbrows
    mesh = plsc.VectorSubcoreMesh(
        core_axis_name="c", subcore_axis_name="s", num_cores=NCORE, num_subcores=NSUB
    )

    @functools.partial(
        pl.kernel,
        out_type=jax.ShapeDtypeStruct((NCORE, NFLAT, 64), jnp.float32),
        mesh=mesh,
        scratch_types=[
            pltpu.VMEM((nbrows, 128), jnp.int32),
            pltpu.VMEM((nbrows, 128), jnp.int32),
            pltpu.VMEM((nbuf, 128, 64), jnp.float32),
            pltpu.SemaphoreType.DMA,
            pltpu.VMEM_SHARED((NFLAT, 64), jnp.float32),
        ],
        compiler_params=pltpu.CompilerParams(use_tc_tiling_on_sc=False),
    )
    def k(u_hbm, src_hbm, dst_hbm, out_hbm, sidx, didx, rows, sem, acc):
        c = lax.axis_index("c")
        s = lax.axis_index("s")
        coff = c * NFLAT

        pltpu.sync_copy(
            u_hbm.at[pl.ds(coff + s * nf_sub, nf_sub)],
            acc.at[pl.ds(s * nf_sub, nf_sub)],
        )
        base = s * rows_per_sub
        plsc.subcore_barrier()

        def blk(bi, carry0):
            rb = base + bi * nbrows
            pltpu.sync_copy(src_hbm.at[pl.ds(rb, nbrows)], sidx)
            pltpu.sync_copy(dst_hbm.at[pl.ds(rb, nbrows)], didx)

            def adj(j, carry):
                for q in range(8):
                    sl = pl.ds(q * 16, 16)
                    sidx[j, sl] = sidx[j, sl] + coff
                return carry

            lax.fori_loop(0, nbrows, adj, 0)

            pltpu.async_copy(u_hbm.at[sidx.at[0]], rows.at[0], sem)

            def step(j, carry):
                pltpu.make_async_copy(
                    u_hbm.at[sidx.at[j]], rows.at[j % nbuf], sem
                ).wait()

                @pl.when(j + 1 < nbrows)
                def _():
                    pltpu.async_copy(
                        u_hbm.at[sidx.at[j + 1]], rows.at[(j + 1) % nbuf], sem
                    )

                pltpu.sync_copy(rows.at[j % nbuf], acc.at[didx.at[j]], add=True)
                return carry

            lax.fori_loop(0, nbrows, step, 0)
            return carry0

        lax.fori_loop(0, nblk, blk, 0)
        plsc.subcore_barrier()
        pltpu.sync_copy(
            acc.at[pl.ds(s * nf_sub, nf_sub)],
            out_hbm.at[c, pl.ds(s * nf_sub, nf_sub)],
        )

    return k(u2, src2, dst2)


def _tc_graph_fuse(graph_in, wf_in, w_in, graph_out, wf_out, w_out):
    def body(gi, fi, wi, go, fo, wo, oi, oo, ai, ao):
        g = pl.program_id(0)

        @pl.when(g == 0)
        def _():
            ai[...] = jnp.zeros_like(ai)
            ao[...] = jnp.zeros_like(ao)

        ai[...] += gi[0] * fi[0]
        ao[...] += go[0] * fo[0]

        @pl.when(g == NG - 1)
        def _():
            oi[...] = jnp.dot(ai[...], wi[...], preferred_element_type=jnp.float32)
            oo[...] = jnp.dot(ao[...], wo[...], preferred_element_type=jnp.float32)

    gspec = pl.BlockSpec((1, NST, NST), lambda g: (g, 0, 0))
    wspec = pl.BlockSpec((NST, NST), lambda g: (0, 0))
    return pl.pallas_call(
        body,
        grid=(NG,),
        in_specs=[gspec, gspec, wspec, gspec, gspec, wspec],
        out_specs=[wspec, wspec],
        out_shape=[jax.ShapeDtypeStruct((NST, NST), jnp.float32)] * 2,
        scratch_shapes=[
            pltpu.VMEM((NST, NST), jnp.float32),
            pltpu.VMEM((NST, NST), jnp.float32),
        ],
    )(graph_in, wf_in, w_in, graph_out, wf_out, w_out)


def _tc_mgc_batch(g_in, g_out, x, b_in, b_out, tw, dinv, rw=None, rb=None):
    ci = x.shape[-1]
    has_res = rw is not None

    def body(*refs):
        if has_res:
            gi, go, xr, bi, bo, twa, twb, dv, rwr, rbr, ur, rr = refs
        else:
            gi, go, xr, bi, bo, twa, twb, dv, ur = refs
        xb = xr[0]
        yin = jax.nn.relu(
            jnp.dot(gi[...], xb, preferred_element_type=jnp.float32) + bi[...]
        )
        yout = jax.nn.relu(
            jnp.dot(go[...], xb, preferred_element_type=jnp.float32) + bo[...]
        )
        h = jnp.dot(yin, twa[...], preferred_element_type=jnp.float32) + jnp.dot(
            yout, twb[...], preferred_element_type=jnp.float32
        )
        u = h * dv[...]
        ur[0, 0] = u[:, :COUT]
        ur[1, 0] = u[:, COUT:]
        if has_res:
            rr[0] = jax.nn.relu(
                jnp.dot(xb, rwr[...], preferred_element_type=jnp.float32) + rbr[...]
            )

    full = lambda shp: pl.BlockSpec(shp, lambda b: tuple(0 for _ in shp))
    in_specs = [
        full((NST, NST)),
        full((NST, NST)),
        pl.BlockSpec((1, NST, ci), lambda b: (b, 0, 0)),
        full((1, ci)),
        full((1, ci)),
        full((ci, 2 * COUT)),
        full((ci, 2 * COUT)),
        pl.BlockSpec((NST, 1), lambda b: (b, 0)),
    ]
    out_shape = [jax.ShapeDtypeStruct((2, BATCH, NST, COUT), jnp.float32)]
    out_specs = [pl.BlockSpec((2, 1, NST, COUT), lambda b: (0, b, 0, 0))]
    args = [
        g_in, g_out, x, b_in.reshape(1, ci), b_out.reshape(1, ci),
        tw[:ci], tw[ci:], dinv,
    ]
    if has_res:
        in_specs += [full((ci, COUT)), full((1, COUT))]
        out_shape.append(jax.ShapeDtypeStruct((BATCH, NST, COUT), jnp.float32))
        out_specs.append(pl.BlockSpec((1, NST, COUT), lambda b: (b, 0, 0)))
        args += [rw, rb.reshape(1, COUT)]
    res = pl.pallas_call(
        body,
        grid=(BATCH,),
        in_specs=in_specs,
        out_specs=out_specs,
        out_shape=out_shape,
    )(*args)
    return res if has_res else (res[0], None)


def _tc_glu(s_agg, dinv2, tb, res, next_w=None, dinv_next=None):
    nb = 16
    rows = NFLAT // nb
    has_next = next_w is not None

    def body(*refs):
        if has_next:
            sr, dv, tbr, rr, nw, dvn, zr, unr = refs
        else:
            sr, dv, tbr, rr, zr = refs
        d = dv[...]
        a = sr[0] * d + tbr[0, :COUT]
        g = sr[1] * d + tbr[0, COUT:]
        z = a * jax.nn.sigmoid(g) + rr[...]
        zr[...] = z
        if has_next:
            un = jnp.dot(z, nw[...], preferred_element_type=jnp.float32) * dvn[...]
            unr[0] = un[:, :COUT]
            unr[1] = un[:, COUT:]

    in_specs = [
        pl.BlockSpec((2, rows, COUT), lambda i: (0, i, 0)),
        pl.BlockSpec((rows, 1), lambda i: (i, 0)),
        pl.BlockSpec((1, 2 * COUT), lambda i: (0, 0)),
        pl.BlockSpec((rows, COUT), lambda i: (i, 0)),
    ]
    out_shape = [jax.ShapeDtypeStruct((NFLAT, COUT), jnp.float32)]
    out_specs = [pl.BlockSpec((rows, COUT), lambda i: (i, 0))]
    args = [s_agg, dinv2, tb.reshape(1, 2 * COUT), res]
    if has_next:
        in_specs += [
            pl.BlockSpec((COUT, 2 * COUT), lambda i: (0, 0)),
            pl.BlockSpec((rows, 1), lambda i: (i, 0)),
        ]
        out_shape.append(jax.ShapeDtypeStruct((2, NFLAT, COUT), jnp.float32))
        out_specs.append(pl.BlockSpec((2, rows, COUT), lambda i: (0, i, 0)))
        args += [next_w, dinv_next]
    out = pl.pallas_call(
        body,
        grid=(nb,),
        in_specs=in_specs,
        out_specs=out_specs,
        out_shape=out_shape,
    )(*args)
    return out if has_next else (out[0], None)


def _tc_dinv(deg_s, deg_l):
    def body(ds_, dl_, os_, ol_):
        os_[...] = lax.rsqrt(1.0 + ds_[0, :, :1] + ds_[1, :, :1])
        ol_[...] = lax.rsqrt(1.0 + dl_[0, :, :1] + dl_[1, :, :1])

    spec = pl.BlockSpec((2, NFLAT, 8), lambda: (0, 0, 0))
    ospec = pl.BlockSpec((NFLAT, 1), lambda: (0, 0))
    return pl.pallas_call(
        body,
        grid=(),
        in_specs=[spec, spec],
        out_specs=[ospec, ospec],
        out_shape=[jax.ShapeDtypeStruct((NFLAT, 1), jnp.float32)] * 2,
    )(deg_s, deg_l)


def _tc_fusion(xn, xp, xt, fw, fb, lt0_w, dinv_l):
    nb = 16
    rows = NFLAT // nb

    def body(xnr, xpr, xtr, fw0, fw1, fw2, fbr, nwr, dvr, hr, ur):
        h = (
            jnp.dot(xnr[...], fw0[...], preferred_element_type=jnp.float32)
            + jnp.dot(xpr[...], fw1[...], preferred_element_type=jnp.float32)
            + jnp.dot(xtr[...], fw2[...], preferred_element_type=jnp.float32)
            + fbr[...]
        )
        h = jax.nn.relu(h)
        hr[...] = h
        u = jnp.dot(h, nwr[...], preferred_element_type=jnp.float32) * dvr[...]
        ur[0] = u[:, :COUT]
        ur[1] = u[:, COUT:]

    rs = lambda: pl.BlockSpec((rows, COUT), lambda i: (i, 0))
    ws = lambda: pl.BlockSpec((COUT, COUT), lambda i: (0, 0))
    return pl.pallas_call(
        body,
        grid=(nb,),
        in_specs=[
            rs(),
            rs(),
            rs(),
            ws(),
            ws(),
            ws(),
            pl.BlockSpec((1, COUT), lambda i: (0, 0)),
            pl.BlockSpec((COUT, 2 * COUT), lambda i: (0, 0)),
            pl.BlockSpec((rows, 1), lambda i: (i, 0)),
        ],
        out_specs=[rs(), pl.BlockSpec((2, rows, COUT), lambda i: (0, i, 0))],
        out_shape=[
            jax.ShapeDtypeStruct((NFLAT, COUT), jnp.float32),
            jax.ShapeDtypeStruct((2, NFLAT, COUT), jnp.float32),
        ],
    )(
        xn, xp, xt, fw[:COUT], fw[COUT : 2 * COUT], fw[2 * COUT :],
        fb.reshape(1, COUT), lt0_w, dinv_l,
    )


def _tc_head(h, x_fc, alive, p):
    nb = 16
    rows = NFLAT // nb
    bscale = (1.0 + 1e-05) ** -0.5

    def body(hr, xfr, ar, w0a, w0b, b0, g0, be0, w1, b1, g1, be1, w2, b2, g2, be2, w3, b3, outr):
        m = (ar[...] == 1).astype(jnp.float32)
        hm = hr[...] * m
        xm = xfr[...] * m
        t = (
            jnp.dot(hm, w0a[...], preferred_element_type=jnp.float32)
            + jnp.dot(xm, w0b[...], preferred_element_type=jnp.float32)
            + b0[...]
        )
        t = jax.nn.relu(t * bscale * g0[...] + be0[...])
        t = jnp.dot(t, w1[...], preferred_element_type=jnp.float32) + b1[...]
        t = jax.nn.relu(t * bscale * g1[...] + be1[...])
        t = jnp.dot(t, w2[...], preferred_element_type=jnp.float32) + b2[...]
        t = jax.nn.relu(t * bscale * g2[...] + be2[...])
        t = jnp.dot(t, w3[...], preferred_element_type=jnp.float32) + b3[...]
        outr[...] = jax.nn.relu(t)

    full = lambda shp: pl.BlockSpec(shp, lambda i: tuple(0 for _ in shp))
    in_specs = [
        pl.BlockSpec((rows, COUT), lambda i: (i, 0)),
        pl.BlockSpec((rows, 8), lambda i: (i, 0)),
        pl.BlockSpec((rows, 1), lambda i: (i, 0)),
    ]
    args = [h, x_fc, alive.reshape(NFLAT, 1)]
    dims = [(COUT + 8, 64), (64, 32), (32, 16), (16, 2)]
    for i, (a, b) in enumerate(dims):
        if i == 0:
            in_specs += [full((COUT, b)), full((8, b)), full((1, b))]
            args += [
                p["fc0_w"][:COUT],
                p["fc0_w"][COUT:],
                p["fc0_b"].reshape(1, b),
            ]
        else:
            in_specs += [full((a, b)), full((1, b))]
            args += [p["fc%d_w" % i], p["fc%d_b" % i].reshape(1, b)]
        if i < 3:
            in_specs += [full((1, b)), full((1, b))]
            args += [p["bn%d_g" % i].reshape(1, b), p["bn%d_b" % i].reshape(1, b)]
    return pl.pallas_call(
        body,
        grid=(nb,),
        in_specs=in_specs,
        out_specs=pl.BlockSpec((rows, 2), lambda i: (i, 0)),
        out_shape=jax.ShapeDtypeStruct((NFLAT, 2), jnp.float32),
    )(*args)


def _stconv(x, g_pair, ei2, dinv2, p, pfx, first):
    g_in, g_out = g_pair
    rw = p[pfx + "rw"] if first else None
    rb = p[pfx + "rb"] if first else None
    u, res = _tc_mgc_batch(
        g_in, g_out, x, p[pfx + "b_in"], p[pfx + "b_out"], p[pfx + "tw"],
        dinv2, rw, rb,
    )
    if not first:
        res = x.reshape(NFLAT, COUT)
    else:
        res = res.reshape(NFLAT, COUT)
    u2 = u.reshape(2 * NFLAT, COUT)
    s_agg = _sc_gcn_aggregate(u2, ei2[0], ei2[1]).reshape(2, NFLAT, COUT)
    z, _ = _tc_glu(s_agg, dinv2, p[pfx + "tb"], res)
    return z.reshape(BATCH, NST, COUT)


def kernel(x, x_fc, is_alive, graph_in, graph_out, edge_index_short, edge_index_log, params):
    p = params
    es2 = [
        edge_index_short[i].reshape(-1, 128).astype(jnp.int32) for i in (0, 1)
    ]
    el2 = [edge_index_log[i].reshape(-1, 128).astype(jnp.int32) for i in (0, 1)]
    zeros_pad = jnp.zeros((NFLAT, 8), jnp.float32)
    ones_pad = jnp.concatenate(
        [jnp.ones((128, 1), jnp.float32), jnp.zeros((128, 7), jnp.float32)], axis=1
    )

    deg_s = _sc_degree(es2[1], zeros_pad, ones_pad)
    deg_l = _sc_degree(el2[1], zeros_pad, ones_pad)
    dinv_s, dinv_l = _tc_dinv(deg_s, deg_l)

    xs = {
        "now": x[:, :, :CIN],
        "period": x[:, :, CIN:-CIN],
        "trend": x[:, :, -CIN:],
    }
    outs = {}
    for name in ("now", "period", "trend"):
        h = xs[name]
        for li in range(2):
            pfx = name + str(li) + "_"
            g_pair = _tc_graph_fuse(
                graph_in, p[pfx + "wf_in"], p[pfx + "w_in"],
                graph_out, p[pfx + "wf_out"], p[pfx + "w_out"],
            )
            h = _stconv(h, g_pair, es2, dinv_s, p, pfx, li == 0)
        outs[name] = h.reshape(NFLAT, COUT)

    h, u = _tc_fusion(
        outs["now"], outs["period"], outs["trend"],
        p["fusion_w"], p["fusion_b"], p["lt0_w"], dinv_l,
    )
    for i in range(3):
        u2 = u.reshape(2 * NFLAT, COUT)
        s_agg = _sc_gcn_aggregate(u2, el2[0], el2[1]).reshape(2, NFLAT, COUT)
        nw = p["lt%d_w" % (i + 1)] if i < 2 else None
        dn = dinv_l if i < 2 else None
        h, u = _tc_glu(s_agg, dinv_l, p["lt%d_b" % i], h, nw, dn)

    return _tc_head(h, x_fc, is_alive, p)


# speedup vs baseline: 20.2751x; 1.3988x over previous
import functools
import jax
import jax.numpy as jnp
from jax import lax
from jax.experimental import pallas as pl
from jax.experimental.pallas import tpu as pltpu
from jax.experimental.pallas import tpu_sc as plsc

CIN = 16
COUT = 64
NST = 512
BATCH = 32
NG = 7
NFLAT = BATCH * NST
NSUB = 16
NCORE = 2


def _sc_degree(dst2, zeros_pad, ones_pad):
    nrows = dst2.shape[0]
    rows_per_sub = nrows // (NCORE * NSUB)
    nf_sub = NFLAT // NSUB
    mesh = plsc.VectorSubcoreMesh(
        core_axis_name="c", subcore_axis_name="s", num_cores=NCORE, num_subcores=NSUB
    )

    @functools.partial(
        pl.kernel,
        out_type=jax.ShapeDtypeStruct((NCORE, NFLAT, 8), jnp.float32),
        mesh=mesh,
        scratch_types=[
            pltpu.VMEM((rows_per_sub, 128), jnp.int32),
            pltpu.VMEM((128, 8), jnp.float32),
            pltpu.VMEM_SHARED((NFLAT, 8), jnp.float32),
            pltpu.SemaphoreType.DMA,
        ],
        compiler_params=pltpu.CompilerParams(use_tc_tiling_on_sc=False),
    )
    def k(dst_hbm, z_hbm, one_hbm, out_hbm, idx_v, ones_v, acc, sem):
        c = lax.axis_index("c")
        s = lax.axis_index("s")
        pltpu.sync_copy(
            z_hbm.at[pl.ds(s * nf_sub, nf_sub)],
            acc.at[pl.ds(s * nf_sub, nf_sub)],
        )
        pltpu.sync_copy(one_hbm, ones_v)
        base = (c * NSUB + s) * rows_per_sub
        pltpu.sync_copy(dst_hbm.at[pl.ds(base, rows_per_sub)], idx_v)
        plsc.subcore_barrier()

        def step(j, carry):
            pltpu.async_copy(ones_v, acc.at[idx_v.at[j]], sem, add=True)
            return carry

        lax.fori_loop(0, rows_per_sub, step, 0)

        def drain(j, carry):
            pltpu.make_async_copy(ones_v, acc.at[idx_v.at[j]], sem).wait()
            return carry

        lax.fori_loop(0, rows_per_sub, drain, 0)
        plsc.subcore_barrier()
        pltpu.sync_copy(
            acc.at[pl.ds(s * nf_sub, nf_sub)],
            out_hbm.at[c, pl.ds(s * nf_sub, nf_sub)],
        )

    return k(dst2, zeros_pad, ones_pad)


def _sc_gcn_aggregate(u2, src2, dst2):
    nrows = src2.shape[0]
    rows_per_sub = nrows // NSUB
    nf_sub = NFLAT // NSUB
    nbuf = 4
    nbrows = 32
    nblk = rows_per_sub // nbrows
    mesh = plsc.VectorSubcoreMesh(
        core_axis_name="c", subcore_axis_name="s", num_cores=NCORE, num_subcores=NSUB
    )

    @functools.partial(
        pl.kernel,
        out_type=jax.ShapeDtypeStruct((NCORE, NFLAT, 64), jnp.float32),
        mesh=mesh,
        scratch_types=[
            pltpu.VMEM((nbrows, 128), jnp.int32),
            pltpu.VMEM((nbrows, 128), jnp.int32),
            pltpu.VMEM((nbuf, 128, 64), jnp.float32),
            pltpu.SemaphoreType.DMA,
            pltpu.SemaphoreType.DMA,
            pltpu.VMEM_SHARED((NFLAT, 64), jnp.float32),
        ],
        compiler_params=pltpu.CompilerParams(use_tc_tiling_on_sc=False),
    )
    def k(u_hbm, src_hbm, dst_hbm, out_hbm, sidx, didx, rows, sem_g, sem_s, acc):
        c = lax.axis_index("c")
        s = lax.axis_index("s")
        coff = c * NFLAT

        pltpu.sync_copy(
            u_hbm.at[pl.ds(coff + s * nf_sub, nf_sub)],
            acc.at[pl.ds(s * nf_sub, nf_sub)],
        )
        base = s * rows_per_sub
        plsc.subcore_barrier()

        def blk(bi, carry0):
            rb = base + bi * nbrows
            pltpu.sync_copy(src_hbm.at[pl.ds(rb, nbrows)], sidx)
            pltpu.sync_copy(dst_hbm.at[pl.ds(rb, nbrows)], didx)

            def adj(j, carry):
                for q in range(8):
                    sl = pl.ds(q * 16, 16)
                    sidx[j, sl] = sidx[j, sl] + coff
                return carry

            lax.fori_loop(0, nbrows, adj, 0)

            pltpu.async_copy(u_hbm.at[sidx.at[0]], rows.at[0], sem_g)
            pltpu.async_copy(u_hbm.at[sidx.at[1]], rows.at[1], sem_g)

            def step(j, carry):
                pltpu.make_async_copy(
                    u_hbm.at[sidx.at[j]], rows.at[j % nbuf], sem_g
                ).wait()
                pltpu.async_copy(
                    rows.at[j % nbuf], acc.at[didx.at[j]], sem_s, add=True
                )

                @pl.when(j + 2 < nbrows)
                def _():
                    @pl.when(j >= 2)
                    def _():
                        pltpu.make_async_copy(
                            rows.at[j % nbuf], acc.at[didx.at[j]], sem_s
                        ).wait()

                    pltpu.async_copy(
                        u_hbm.at[sidx.at[j + 2]], rows.at[(j + 2) % nbuf], sem_g
                    )

                return carry

            lax.fori_loop(0, nbrows, step, 0)

            def drain(j, carry):
                pltpu.make_async_copy(rows.at[0], acc.at[didx.at[0]], sem_s).wait()
                return carry

            lax.fori_loop(0, 4, drain, 0)
            return carry0

        lax.fori_loop(0, nblk, blk, 0)
        plsc.subcore_barrier()
        pltpu.sync_copy(
            acc.at[pl.ds(s * nf_sub, nf_sub)],
            out_hbm.at[c, pl.ds(s * nf_sub, nf_sub)],
        )

    return k(u2, src2, dst2)


def _tc_graph_fuse(graph_in, wf_in, w_in, graph_out, wf_out, w_out):
    def body(gi, fi, wi, go, fo, wo, oi, oo, ai, ao):
        g = pl.program_id(0)

        @pl.when(g == 0)
        def _():
            ai[...] = jnp.zeros_like(ai)
            ao[...] = jnp.zeros_like(ao)

        ai[...] += gi[0] * fi[0]
        ao[...] += go[0] * fo[0]

        @pl.when(g == NG - 1)
        def _():
            oi[...] = jnp.dot(ai[...], wi[...], preferred_element_type=jnp.float32)
            oo[...] = jnp.dot(ao[...], wo[...], preferred_element_type=jnp.float32)

    gspec = pl.BlockSpec((1, NST, NST), lambda g: (g, 0, 0))
    wspec = pl.BlockSpec((NST, NST), lambda g: (0, 0))
    return pl.pallas_call(
        body,
        grid=(NG,),
        in_specs=[gspec, gspec, wspec, gspec, gspec, wspec],
        out_specs=[wspec, wspec],
        out_shape=[jax.ShapeDtypeStruct((NST, NST), jnp.float32)] * 2,
        scratch_shapes=[
            pltpu.VMEM((NST, NST), jnp.float32),
            pltpu.VMEM((NST, NST), jnp.float32),
        ],
    )(graph_in, wf_in, w_in, graph_out, wf_out, w_out)


def _tc_mgc_batch(g_in, g_out, x, b_in, b_out, tw, dinv, rw=None, rb=None):
    ci = x.shape[-1]
    has_res = rw is not None

    def body(*refs):
        if has_res:
            gi, go, xr, bi, bo, twa, twb, dv, rwr, rbr, ur, rr = refs
        else:
            gi, go, xr, bi, bo, twa, twb, dv, ur = refs
        xb = xr[0]
        yin = jax.nn.relu(
            jnp.dot(gi[...], xb, preferred_element_type=jnp.float32) + bi[...]
        )
        yout = jax.nn.relu(
            jnp.dot(go[...], xb, preferred_element_type=jnp.float32) + bo[...]
        )
        h = jnp.dot(yin, twa[...], preferred_element_type=jnp.float32) + jnp.dot(
            yout, twb[...], preferred_element_type=jnp.float32
        )
        u = h * dv[...]
        ur[0, 0] = u[:, :COUT]
        ur[1, 0] = u[:, COUT:]
        if has_res:
            rr[0] = jax.nn.relu(
                jnp.dot(xb, rwr[...], preferred_element_type=jnp.float32) + rbr[...]
            )

    full = lambda shp: pl.BlockSpec(shp, lambda b: tuple(0 for _ in shp))
    in_specs = [
        full((NST, NST)),
        full((NST, NST)),
        pl.BlockSpec((1, NST, ci), lambda b: (b, 0, 0)),
        full((1, ci)),
        full((1, ci)),
        full((ci, 2 * COUT)),
        full((ci, 2 * COUT)),
        pl.BlockSpec((NST, 1), lambda b: (b, 0)),
    ]
    out_shape = [jax.ShapeDtypeStruct((2, BATCH, NST, COUT), jnp.float32)]
    out_specs = [pl.BlockSpec((2, 1, NST, COUT), lambda b: (0, b, 0, 0))]
    args = [
        g_in, g_out, x, b_in.reshape(1, ci), b_out.reshape(1, ci),
        tw[:ci], tw[ci:], dinv,
    ]
    if has_res:
        in_specs += [full((ci, COUT)), full((1, COUT))]
        out_shape.append(jax.ShapeDtypeStruct((BATCH, NST, COUT), jnp.float32))
        out_specs.append(pl.BlockSpec((1, NST, COUT), lambda b: (b, 0, 0)))
        args += [rw, rb.reshape(1, COUT)]
    res = pl.pallas_call(
        body,
        grid=(BATCH,),
        in_specs=in_specs,
        out_specs=out_specs,
        out_shape=out_shape,
    )(*args)
    return res if has_res else (res[0], None)


def _tc_glu(s_agg, dinv2, tb, res, next_w=None, dinv_next=None):
    nb = 16
    rows = NFLAT // nb
    has_next = next_w is not None

    def body(*refs):
        if has_next:
            sr, dv, tbr, rr, nw, dvn, zr, unr = refs
        else:
            sr, dv, tbr, rr, zr = refs
        d = dv[...]
        a = sr[0] * d + tbr[0, :COUT]
        g = sr[1] * d + tbr[0, COUT:]
        z = a * jax.nn.sigmoid(g) + rr[...]
        zr[...] = z
        if has_next:
            un = jnp.dot(z, nw[...], preferred_element_type=jnp.float32) * dvn[...]
            unr[0] = un[:, :COUT]
            unr[1] = un[:, COUT:]

    in_specs = [
        pl.BlockSpec((2, rows, COUT), lambda i: (0, i, 0)),
        pl.BlockSpec((rows, 1), lambda i: (i, 0)),
        pl.BlockSpec((1, 2 * COUT), lambda i: (0, 0)),
        pl.BlockSpec((rows, COUT), lambda i: (i, 0)),
    ]
    out_shape = [jax.ShapeDtypeStruct((NFLAT, COUT), jnp.float32)]
    out_specs = [pl.BlockSpec((rows, COUT), lambda i: (i, 0))]
    args = [s_agg, dinv2, tb.reshape(1, 2 * COUT), res]
    if has_next:
        in_specs += [
            pl.BlockSpec((COUT, 2 * COUT), lambda i: (0, 0)),
            pl.BlockSpec((rows, 1), lambda i: (i, 0)),
        ]
        out_shape.append(jax.ShapeDtypeStruct((2, NFLAT, COUT), jnp.float32))
        out_specs.append(pl.BlockSpec((2, rows, COUT), lambda i: (0, i, 0)))
        args += [next_w, dinv_next]
    out = pl.pallas_call(
        body,
        grid=(nb,),
        in_specs=in_specs,
        out_specs=out_specs,
        out_shape=out_shape,
    )(*args)
    return out if has_next else (out[0], None)


def _tc_dinv(deg_s, deg_l):
    def body(ds_, dl_, os_, ol_):
        os_[...] = lax.rsqrt(1.0 + ds_[0, :, :1] + ds_[1, :, :1])
        ol_[...] = lax.rsqrt(1.0 + dl_[0, :, :1] + dl_[1, :, :1])

    spec = pl.BlockSpec((2, NFLAT, 8), lambda: (0, 0, 0))
    ospec = pl.BlockSpec((NFLAT, 1), lambda: (0, 0))
    return pl.pallas_call(
        body,
        grid=(),
        in_specs=[spec, spec],
        out_specs=[ospec, ospec],
        out_shape=[jax.ShapeDtypeStruct((NFLAT, 1), jnp.float32)] * 2,
    )(deg_s, deg_l)


def _tc_fusion(xn, xp, xt, fw, fb, lt0_w, dinv_l):
    nb = 16
    rows = NFLAT // nb

    def body(xnr, xpr, xtr, fw0, fw1, fw2, fbr, nwr, dvr, hr, ur):
        h = (
            jnp.dot(xnr[...], fw0[...], preferred_element_type=jnp.float32)
            + jnp.dot(xpr[...], fw1[...], preferred_element_type=jnp.float32)
            + jnp.dot(xtr[...], fw2[...], preferred_element_type=jnp.float32)
            + fbr[...]
        )
        h = jax.nn.relu(h)
        hr[...] = h
        u = jnp.dot(h, nwr[...], preferred_element_type=jnp.float32) * dvr[...]
        ur[0] = u[:, :COUT]
        ur[1] = u[:, COUT:]

    rs = lambda: pl.BlockSpec((rows, COUT), lambda i: (i, 0))
    ws = lambda: pl.BlockSpec((COUT, COUT), lambda i: (0, 0))
    return pl.pallas_call(
        body,
        grid=(nb,),
        in_specs=[
            rs(),
            rs(),
            rs(),
            ws(),
            ws(),
            ws(),
            pl.BlockSpec((1, COUT), lambda i: (0, 0)),
            pl.BlockSpec((COUT, 2 * COUT), lambda i: (0, 0)),
            pl.BlockSpec((rows, 1), lambda i: (i, 0)),
        ],
        out_specs=[rs(), pl.BlockSpec((2, rows, COUT), lambda i: (0, i, 0))],
        out_shape=[
            jax.ShapeDtypeStruct((NFLAT, COUT), jnp.float32),
            jax.ShapeDtypeStruct((2, NFLAT, COUT), jnp.float32),
        ],
    )(
        xn, xp, xt, fw[:COUT], fw[COUT : 2 * COUT], fw[2 * COUT :],
        fb.reshape(1, COUT), lt0_w, dinv_l,
    )


def _tc_head(h, x_fc, alive, p):
    nb = 16
    rows = NFLAT // nb
    bscale = (1.0 + 1e-05) ** -0.5

    def body(hr, xfr, ar, w0a, w0b, b0, g0, be0, w1, b1, g1, be1, w2, b2, g2, be2, w3, b3, outr):
        m = (ar[...] == 1).astype(jnp.float32)
        hm = hr[...] * m
        xm = xfr[...] * m
        t = (
            jnp.dot(hm, w0a[...], preferred_element_type=jnp.float32)
            + jnp.dot(xm, w0b[...], preferred_element_type=jnp.float32)
            + b0[...]
        )
        t = jax.nn.relu(t * bscale * g0[...] + be0[...])
        t = jnp.dot(t, w1[...], preferred_element_type=jnp.float32) + b1[...]
        t = jax.nn.relu(t * bscale * g1[...] + be1[...])
        t = jnp.dot(t, w2[...], preferred_element_type=jnp.float32) + b2[...]
        t = jax.nn.relu(t * bscale * g2[...] + be2[...])
        t = jnp.dot(t, w3[...], preferred_element_type=jnp.float32) + b3[...]
        outr[...] = jax.nn.relu(t)

    full = lambda shp: pl.BlockSpec(shp, lambda i: tuple(0 for _ in shp))
    in_specs = [
        pl.BlockSpec((rows, COUT), lambda i: (i, 0)),
        pl.BlockSpec((rows, 8), lambda i: (i, 0)),
        pl.BlockSpec((rows, 1), lambda i: (i, 0)),
    ]
    args = [h, x_fc, alive.reshape(NFLAT, 1)]
    dims = [(COUT + 8, 64), (64, 32), (32, 16), (16, 2)]
    for i, (a, b) in enumerate(dims):
        if i == 0:
            in_specs += [full((COUT, b)), full((8, b)), full((1, b))]
            args += [
                p["fc0_w"][:COUT],
                p["fc0_w"][COUT:],
                p["fc0_b"].reshape(1, b),
            ]
        else:
            in_specs += [full((a, b)), full((1, b))]
            args += [p["fc%d_w" % i], p["fc%d_b" % i].reshape(1, b)]
        if i < 3:
            in_specs += [full((1, b)), full((1, b))]
            args += [p["bn%d_g" % i].reshape(1, b), p["bn%d_b" % i].reshape(1, b)]
    return pl.pallas_call(
        body,
        grid=(nb,),
        in_specs=in_specs,
        out_specs=pl.BlockSpec((rows, 2), lambda i: (i, 0)),
        out_shape=jax.ShapeDtypeStruct((NFLAT, 2), jnp.float32),
    )(*args)


def _stconv(x, g_pair, ei2, dinv2, p, pfx, first):
    g_in, g_out = g_pair
    rw = p[pfx + "rw"] if first else None
    rb = p[pfx + "rb"] if first else None
    u, res = _tc_mgc_batch(
        g_in, g_out, x, p[pfx + "b_in"], p[pfx + "b_out"], p[pfx + "tw"],
        dinv2, rw, rb,
    )
    if not first:
        res = x.reshape(NFLAT, COUT)
    else:
        res = res.reshape(NFLAT, COUT)
    u2 = u.reshape(2 * NFLAT, COUT)
    s_agg = _sc_gcn_aggregate(u2, ei2[0], ei2[1]).reshape(2, NFLAT, COUT)
    z, _ = _tc_glu(s_agg, dinv2, p[pfx + "tb"], res)
    return z.reshape(BATCH, NST, COUT)


def kernel(x, x_fc, is_alive, graph_in, graph_out, edge_index_short, edge_index_log, params):
    p = params
    es2 = [
        edge_index_short[i].reshape(-1, 128).astype(jnp.int32) for i in (0, 1)
    ]
    el2 = [edge_index_log[i].reshape(-1, 128).astype(jnp.int32) for i in (0, 1)]
    zeros_pad = jnp.zeros((NFLAT, 8), jnp.float32)
    ones_pad = jnp.concatenate(
        [jnp.ones((128, 1), jnp.float32), jnp.zeros((128, 7), jnp.float32)], axis=1
    )

    deg_s = _sc_degree(es2[1], zeros_pad, ones_pad)
    deg_l = _sc_degree(el2[1], zeros_pad, ones_pad)
    dinv_s, dinv_l = _tc_dinv(deg_s, deg_l)

    xs = {
        "now": x[:, :, :CIN],
        "period": x[:, :, CIN:-CIN],
        "trend": x[:, :, -CIN:],
    }
    outs = {}
    for name in ("now", "period", "trend"):
        h = xs[name]
        for li in range(2):
            pfx = name + str(li) + "_"
            g_pair = _tc_graph_fuse(
                graph_in, p[pfx + "wf_in"], p[pfx + "w_in"],
                graph_out, p[pfx + "wf_out"], p[pfx + "w_out"],
            )
            h = _stconv(h, g_pair, es2, dinv_s, p, pfx, li == 0)
        outs[name] = h.reshape(NFLAT, COUT)

    h, u = _tc_fusion(
        outs["now"], outs["period"], outs["trend"],
        p["fusion_w"], p["fusion_b"], p["lt0_w"], dinv_l,
    )
    for i in range(3):
        u2 = u.reshape(2 * NFLAT, COUT)
        s_agg = _sc_gcn_aggregate(u2, el2[0], el2[1]).reshape(2, NFLAT, COUT)
        nw = p["lt%d_w" % (i + 1)] if i < 2 else None
        dn = dinv_l if i < 2 else None
        h, u = _tc_glu(s_agg, dinv_l, p["lt%d_b" % i], h, nw, dn)

    return _tc_head(h, x_fc, is_alive, p)
